# baseline (speedup 1.0000x reference)
import functools

import jax
import jax.numpy as jnp
from jax import lax
from jax.experimental import pallas as pl
from jax.experimental.pallas import tpu as pltpu
from jax.experimental.pallas import tpu_sc as plsc

VOCAB = 1000000
EMBED = 64
OUT = 2
PAD = 0
L = 200
B = 4096

DPROJ = 16
NC, NS = 2, 16
NW = NC * NS
B_PER_W = B // NW
CB = 16
ROWS = CB * L
NCHUNK = B_PER_W // CB

_BLK = 8192


def _project_body(wt_ref, xt_ref, o0_ref, o1_ref):
    p = jnp.dot(wt_ref[...], xt_ref[...], preferred_element_type=jnp.float32)
    p = jnp.where(
        (pl.program_id(0) == 0)
        & (jax.lax.broadcasted_iota(jnp.int32, p.shape, 1) == PAD),
        0.0, p)
    o0_ref[...] = p[0]
    o1_ref[...] = p[1]


_project = pl.pallas_call(
    _project_body,
    grid=(pl.cdiv(VOCAB, _BLK),),
    in_specs=[
        pl.BlockSpec((8, EMBED), lambda i: (0, 0)),
        pl.BlockSpec((EMBED, _BLK), lambda i: (0, i)),
    ],
    out_specs=[
        pl.BlockSpec((_BLK,), lambda i: (i,)),
        pl.BlockSpec((_BLK,), lambda i: (i,)),
    ],
    out_shape=[
        jax.ShapeDtypeStruct((VOCAB,), jnp.float32),
        jax.ShapeDtypeStruct((VOCAB,), jnp.float32),
    ],
)

_mesh = plsc.VectorSubcoreMesh(core_axis_name="c", subcore_axis_name="s")

CI = 1952
NCI = 16
W_ROWS = CI * NCI
TAIL = VOCAB - NW * W_ROWS


@functools.partial(
    pl.kernel,
    mesh=_mesh,
    compiler_params=pltpu.CompilerParams(use_tc_tiling_on_sc=False),
    out_type=jax.ShapeDtypeStruct((VOCAB, DPROJ), jnp.float32),
    scratch_types=[
        pltpu.VMEM((CI, DPROJ), jnp.float32),
        pltpu.VMEM((TAIL, DPROJ), jnp.float32),
    ],
)
def _interleave(p0_hbm, p1_hbm, q_hbm, qb, qt):
    wid = lax.axis_index("s") * NC + lax.axis_index("c")
    base = wid * W_ROWS
    zero_row = jnp.zeros((DPROJ,), jnp.float32)

    def zbody(i, _):
        qb[i, :] = zero_row
        return 0
    lax.fori_loop(0, CI, zbody, 0)

    for c in range(NCI):
        off = base + c * CI
        pltpu.sync_copy(p0_hbm.at[pl.ds(off, CI)], qb.at[:, pl.ds(0, 1)])
        pltpu.sync_copy(p1_hbm.at[pl.ds(off, CI)], qb.at[:, pl.ds(1, 1)])
        pltpu.sync_copy(qb, q_hbm.at[pl.ds(off, CI)])

    @pl.when(wid == 0)
    def _tail():
        def ztail(i, _):
            qt[i, :] = zero_row
            return 0
        lax.fori_loop(0, TAIL, ztail, 0)
        off = NW * W_ROWS
        pltpu.sync_copy(p0_hbm.at[pl.ds(off, TAIL)], qt.at[:, pl.ds(0, 1)])
        pltpu.sync_copy(p1_hbm.at[pl.ds(off, TAIL)], qt.at[:, pl.ds(1, 1)])
        pltpu.sync_copy(qt, q_hbm.at[pl.ds(off, TAIL)])


@functools.partial(
    pl.kernel,
    mesh=_mesh,
    compiler_params=pltpu.CompilerParams(use_tc_tiling_on_sc=False),
    out_type=jax.ShapeDtypeStruct((B, DPROJ), jnp.float32),
    scratch_types=[
        pltpu.VMEM((B_PER_W * L,), jnp.int32),
        pltpu.VMEM((2, ROWS, DPROJ), jnp.float32),
        pltpu.VMEM((CB, DPROJ), jnp.float32),
        pltpu.VMEM((DPROJ,), jnp.float32),
        pltpu.SemaphoreType.DMA,
        pltpu.SemaphoreType.DMA,
    ],
)
def _pool(idx_hbm, proj_hbm, bias_hbm, out_hbm,
          idx_v, rows_v, out_v, bias_v, sem0, sem1):
    wid = lax.axis_index("s") * NC + lax.axis_index("c")
    tok_base = wid * (B_PER_W * L)
    pltpu.sync_copy(idx_hbm.at[pl.ds(tok_base, B_PER_W * L)], idx_v)
    pltpu.sync_copy(bias_hbm, bias_v)
    bias = bias_v[...]
    scale = jnp.float32(1.0 / L)
    sems = (sem0, sem1)

    copies = [None, None]
    copies[0] = pltpu.async_copy(
        proj_hbm.at[idx_v.at[pl.ds(0, ROWS)]], rows_v.at[0], sems[0])
    for c in range(NCHUNK):
        buf = c % 2
        if c + 1 < NCHUNK:
            nb = (c + 1) % 2
            copies[nb] = pltpu.async_copy(
                proj_hbm.at[idx_v.at[pl.ds((c + 1) * ROWS, ROWS)]],
                rows_v.at[nb], sems[nb])
        copies[buf].wait()
        rows = rows_v.at[buf]

        def bbody(b, _, rows=rows):
            def lbody(j, acc):
                r0 = b * L + j * 8
                for u in range(8):
                    acc = acc + rows[r0 + u, :]
                return acc
            acc = lax.fori_loop(0, L // 8, lbody,
                                jnp.zeros((DPROJ,), jnp.float32))
            out_v[b, :] = acc * scale + bias
            return 0

        lax.fori_loop(0, CB, bbody, 0)
        pltpu.sync_copy(out_v, out_hbm.at[pl.ds(wid * B_PER_W + c * CB, CB)])


def kernel(text, emb_table, fc_w, fc_b):
    idx = text.astype(jnp.int32).T.reshape(-1)
    wt = jnp.zeros((8, EMBED), jnp.float32).at[:OUT, :].set(fc_w)
    p0, p1 = _project(wt, emb_table.T)
    proj = _interleave(p0.reshape(VOCAB, 1), p1.reshape(VOCAB, 1))
    bias16 = jnp.zeros((DPROJ,), jnp.float32).at[:OUT].set(fc_b)
    out16 = _pool(idx, proj, bias16)
    return out16[:, :OUT]


# speedup vs baseline: 3.3750x; 3.3750x over previous
import functools

import jax
import jax.numpy as jnp
from jax import lax
from jax.experimental import pallas as pl
from jax.experimental.pallas import tpu as pltpu
from jax.experimental.pallas import tpu_sc as plsc

VOCAB = 1000000
EMBED = 64
OUT = 2
PAD = 0
L = 200
B = 4096

DPROJ = 16
NC, NS = 2, 16
NW = NC * NS
B_PER_W = B // NW
CB = 16
ROWS = CB * L
NCHUNK = B_PER_W // CB

_BLK = 8192


def _project_body(wt_ref, xt_ref, o_ref):
    p = jnp.dot(wt_ref[...], xt_ref[...], preferred_element_type=jnp.float32)
    p = jnp.where(
        (pl.program_id(0) == 0)
        & (jax.lax.broadcasted_iota(jnp.int32, p.shape, 1) == PAD),
        0.0, p)
    o_ref[...] = p


_project = pl.pallas_call(
    _project_body,
    grid=(pl.cdiv(VOCAB, _BLK),),
    in_specs=[
        pl.BlockSpec((DPROJ, EMBED), lambda i: (0, 0)),
        pl.BlockSpec((EMBED, _BLK), lambda i: (0, i)),
    ],
    out_specs=pl.BlockSpec((DPROJ, _BLK), lambda i: (0, i)),
    out_shape=jax.ShapeDtypeStruct((DPROJ, VOCAB), jnp.float32),
)

_mesh = plsc.VectorSubcoreMesh(core_axis_name="c", subcore_axis_name="s")


@functools.partial(
    pl.kernel,
    mesh=_mesh,
    compiler_params=pltpu.CompilerParams(use_tc_tiling_on_sc=False),
    out_type=jax.ShapeDtypeStruct((B, DPROJ), jnp.float32),
    scratch_types=[
        pltpu.VMEM((B_PER_W * L,), jnp.int32),
        pltpu.VMEM((2, ROWS, DPROJ), jnp.float32),
        pltpu.VMEM((CB, DPROJ), jnp.float32),
        pltpu.VMEM((DPROJ,), jnp.float32),
        pltpu.SemaphoreType.DMA,
        pltpu.SemaphoreType.DMA,
    ],
)
def _pool(idx_hbm, proj_hbm, bias_hbm, out_hbm,
          idx_v, rows_v, out_v, bias_v, sem0, sem1):
    wid = lax.axis_index("s") * NC + lax.axis_index("c")
    tok_base = wid * (B_PER_W * L)
    pltpu.sync_copy(idx_hbm.at[pl.ds(tok_base, B_PER_W * L)], idx_v)
    pltpu.sync_copy(bias_hbm, bias_v)
    bias = bias_v[...]
    scale = jnp.float32(1.0 / L)
    sems = (sem0, sem1)

    copies = [None, None]
    copies[0] = pltpu.async_copy(
        proj_hbm.at[idx_v.at[pl.ds(0, ROWS)]], rows_v.at[0], sems[0])
    for c in range(NCHUNK):
        buf = c % 2
        if c + 1 < NCHUNK:
            nb = (c + 1) % 2
            copies[nb] = pltpu.async_copy(
                proj_hbm.at[idx_v.at[pl.ds((c + 1) * ROWS, ROWS)]],
                rows_v.at[nb], sems[nb])
        copies[buf].wait()
        rows = rows_v.at[buf]

        def bbody(b, _, rows=rows):
            def lbody(j, acc):
                r0 = b * L + j * 8
                for u in range(8):
                    acc = acc + rows[r0 + u, :]
                return acc
            acc = lax.fori_loop(0, L // 8, lbody,
                                jnp.zeros((DPROJ,), jnp.float32))
            out_v[b, :] = acc * scale + bias
            return 0

        lax.fori_loop(0, CB, bbody, 0)
        pltpu.sync_copy(out_v, out_hbm.at[pl.ds(wid * B_PER_W + c * CB, CB)])


def kernel(text, emb_table, fc_w, fc_b):
    idx = text.astype(jnp.int32).T.reshape(-1)
    w16t = jnp.zeros((DPROJ, EMBED), jnp.float32).at[:OUT, :].set(fc_w)
    projt = _project(w16t, emb_table.T)
    proj = projt.T
    bias16 = jnp.zeros((DPROJ,), jnp.float32).at[:OUT].set(fc_b)
    out16 = _pool(idx, proj, bias16)
    return out16[:, :OUT]
